# Initial kernel scaffold; baseline (speedup 1.0000x reference)
import jax
import jax.numpy as jnp
from jax.experimental import pallas as pl


def kernel(trg, src, ft_w, ft_b, sa_wq, sa_bq, sa_wk, sa_bk, sa_wv, sa_bv, sa_wo, sa_bo, ea_wq, ea_bq, ea_wk, ea_bk, ea_wv, ea_bv, ea_wo, ea_bo, pf_w1, pf_b1, pf_w2, pf_b2, ln1_g, ln1_b, ln2_g, ln2_b, ln3_g, ln3_b, fc1_w, fc1_b, fc2_w, fc2_b, fc3_w, fc3_b):
    raise NotImplementedError("write your pallas kernel here")


# speedup vs baseline: 1.8831x; 1.8831x over previous
import jax
import jax.numpy as jnp
from jax.experimental import pallas as pl
from jax.experimental.pallas import tpu as pltpu

EPS = 1e-5
N_HEADS = 8


def _ln(x, g, b):
    m = jnp.mean(x, axis=-1, keepdims=True)
    xc = x - m
    v = jnp.mean(xc * xc, axis=-1, keepdims=True)
    return xc * jax.lax.rsqrt(v + EPS) * g + b


def _mha(q_in, kv_in, wq, bq, wk, bk, wv, bv, wo, bo, attn_ref):
    D = q_in.shape[-1]
    dh = D // N_HEADS
    scale = 1.0 / jnp.sqrt(jnp.float32(dh))
    q = jnp.dot(q_in, wq, preferred_element_type=jnp.float32) + bq
    k = jnp.dot(kv_in, wk, preferred_element_type=jnp.float32) + bk
    v = jnp.dot(kv_in, wv, preferred_element_type=jnp.float32) + bv
    outs = []
    for h in range(N_HEADS):
        qh = q[:, h * dh:(h + 1) * dh]
        kh = k[:, h * dh:(h + 1) * dh]
        vh = v[:, h * dh:(h + 1) * dh]
        s = jax.lax.dot_general(qh, kh, (((1,), (1,)), ((), ())),
                                preferred_element_type=jnp.float32) * scale
        m = jnp.max(s, axis=1, keepdims=True)
        e = jnp.exp(s - m)
        denom = jnp.sum(e, axis=1, keepdims=True)
        p = e / denom
        if attn_ref is not None:
            attn_ref[0, h] = p
        outs.append(jnp.dot(p, vh, preferred_element_type=jnp.float32))
    o = jnp.concatenate(outs, axis=1)
    return jnp.dot(o, wo, preferred_element_type=jnp.float32) + bo


def _make_layer_body(first, last):
    def body(*refs):
        it = iter(refs)
        if first:
            trg_ref = next(it)
            ftw_ref = next(it)
            ftb_ref = next(it)
        else:
            x_ref = next(it)
        src_ref = next(it)
        sa = [next(it) for _ in range(8)]
        ea = [next(it) for _ in range(8)]
        pfw1, pfb1, pfw2, pfb2 = next(it), next(it), next(it), next(it)
        ln1g, ln1b, ln2g, ln2b, ln3g, ln3b = (next(it) for _ in range(6))
        if last:
            fc1w, fc1b, fc2w, fc2b, fc3w, fc3b = (next(it) for _ in range(6))
            attn_ref = next(it)
            pooled_ref = next(it)
            label_ref = next(it)
        else:
            xo_ref = next(it)

        if first:
            x = (jnp.dot(trg_ref[0], ftw_ref[...],
                         preferred_element_type=jnp.float32) + ftb_ref[...])
        else:
            x = x_ref[0]

        s_out = _mha(x, x, sa[0][0], sa[1][0], sa[2][0], sa[3][0],
                     sa[4][0], sa[5][0], sa[6][0], sa[7][0], None)
        x = _ln(x + s_out, ln1g[0], ln1b[0])
        c_out = _mha(x, src_ref[0], ea[0][0], ea[1][0], ea[2][0], ea[3][0],
                     ea[4][0], ea[5][0], ea[6][0], ea[7][0],
                     attn_ref if last else None)
        x = _ln(x + c_out, ln2g[0], ln2b[0])
        f = jnp.maximum(
            jnp.dot(x, pfw1[0], preferred_element_type=jnp.float32) + pfb1[0],
            0.0)
        f = jnp.dot(f, pfw2[0], preferred_element_type=jnp.float32) + pfb2[0]
        x = _ln(x + f, ln3g[0], ln3b[0])

        if last:
            sq = jnp.sum(x * x, axis=1, keepdims=True)
            nrm = jnp.sqrt(sq)
            mx = jnp.max(nrm, axis=0, keepdims=True)
            e = jnp.exp(nrm - mx)
            w = e / jnp.sum(e, axis=0, keepdims=True)
            pooled = jax.lax.dot_general(w, x, (((0,), (0,)), ((), ())),
                                         preferred_element_type=jnp.float32)
            h1 = jnp.maximum(
                jnp.dot(pooled, fc1w[...],
                        preferred_element_type=jnp.float32) + fc1b[...], 0.0)
            h2 = jnp.maximum(
                jnp.dot(h1, fc2w[...],
                        preferred_element_type=jnp.float32) + fc2b[...], 0.0)
            lab = (jnp.dot(h2, fc3w[...],
                           preferred_element_type=jnp.float32) + fc3b[...])
            pooled_ref[0] = pooled
            label_ref[0] = lab
        else:
            xo_ref[0] = x

    return body


def kernel(trg, src, ft_w, ft_b,
           sa_wq, sa_bq, sa_wk, sa_bk, sa_wv, sa_bv, sa_wo, sa_bo,
           ea_wq, ea_bq, ea_wk, ea_bk, ea_wv, ea_bv, ea_wo, ea_bo,
           pf_w1, pf_b1, pf_w2, pf_b2,
           ln1_g, ln1_b, ln2_g, ln2_b, ln3_g, ln3_b,
           fc1_w, fc1_b, fc2_w, fc2_b, fc3_w, fc3_b):
    B, St, LOCAL = trg.shape
    Ss, D = src.shape[1], src.shape[2]
    L = sa_wq.shape[0]
    F = pf_w1.shape[2]
    f32 = jnp.float32

    b3 = lambda a: a.reshape(L, 1, a.shape[-1])
    sa_bq3, sa_bk3, sa_bv3, sa_bo3 = map(b3, (sa_bq, sa_bk, sa_bv, sa_bo))
    ea_bq3, ea_bk3, ea_bv3, ea_bo3 = map(b3, (ea_bq, ea_bk, ea_bv, ea_bo))
    pf_b13, pf_b23 = b3(pf_b1), b3(pf_b2)
    ln1_g3, ln1_b3, ln2_g3, ln2_b3, ln3_g3, ln3_b3 = map(
        b3, (ln1_g, ln1_b, ln2_g, ln2_b, ln3_g, ln3_b))
    ft_b2 = ft_b.reshape(1, D)
    fc1_b2, fc2_b2, fc3_b2 = (fc1_b.reshape(1, -1), fc2_b.reshape(1, -1),
                              fc3_b.reshape(1, -1))

    def wspec(l, shape):
        n = len(shape)
        return pl.BlockSpec((1,) + shape, lambda b, l=l: (l,) + (0,) * n)

    def full(shape):
        n = len(shape)
        return pl.BlockSpec(shape, lambda b: (0,) * n)

    x = None
    for l in range(L):
        first, last = l == 0, l == L - 1
        ins = []
        in_specs = []
        if first:
            ins += [trg, ft_w, ft_b2]
            in_specs += [pl.BlockSpec((1, St, LOCAL), lambda b: (b, 0, 0)),
                         full((LOCAL, D)), full((1, D))]
        else:
            ins += [x]
            in_specs += [pl.BlockSpec((1, St, D), lambda b: (b, 0, 0))]
        ins += [src]
        in_specs += [pl.BlockSpec((1, Ss, D), lambda b: (b, 0, 0))]
        for w_, b_ in ((sa_wq, sa_bq3), (sa_wk, sa_bk3), (sa_wv, sa_bv3),
                       (sa_wo, sa_bo3)):
            ins += [w_, b_]
            in_specs += [wspec(l, (D, D)), wspec(l, (1, D))]
        for w_, b_ in ((ea_wq, ea_bq3), (ea_wk, ea_bk3), (ea_wv, ea_bv3),
                       (ea_wo, ea_bo3)):
            ins += [w_, b_]
            in_specs += [wspec(l, (D, D)), wspec(l, (1, D))]
        ins += [pf_w1, pf_b13, pf_w2, pf_b23]
        in_specs += [wspec(l, (D, F)), wspec(l, (1, F)),
                     wspec(l, (F, D)), wspec(l, (1, D))]
        for p_ in (ln1_g3, ln1_b3, ln2_g3, ln2_b3, ln3_g3, ln3_b3):
            ins += [p_]
            in_specs += [wspec(l, (1, D))]
        if last:
            ins += [fc1_w, fc1_b2, fc2_w, fc2_b2, fc3_w, fc3_b2]
            in_specs += [full(fc1_w.shape), full((1, fc1_w.shape[1])),
                         full(fc2_w.shape), full((1, fc2_w.shape[1])),
                         full(fc3_w.shape), full((1, fc3_w.shape[1]))]
            out_shape = [jax.ShapeDtypeStruct((B, N_HEADS, St, Ss), f32),
                         jax.ShapeDtypeStruct((B, 1, D), f32),
                         jax.ShapeDtypeStruct((B, 1, 2), f32)]
            out_specs = [pl.BlockSpec((1, N_HEADS, St, Ss),
                                      lambda b: (b, 0, 0, 0)),
                         pl.BlockSpec((1, 1, D), lambda b: (b, 0, 0)),
                         pl.BlockSpec((1, 1, 2), lambda b: (b, 0, 0))]
        else:
            out_shape = jax.ShapeDtypeStruct((B, St, D), f32)
            out_specs = pl.BlockSpec((1, St, D), lambda b: (b, 0, 0))

        res = pl.pallas_call(
            _make_layer_body(first, last),
            grid=(B,),
            in_specs=in_specs,
            out_specs=out_specs,
            out_shape=out_shape,
            compiler_params=pltpu.CompilerParams(
                dimension_semantics=("parallel",),
                vmem_limit_bytes=56 * 1024 * 1024,
            ),
        )(*ins)
        if last:
            attn, pooled3, label3 = res
        else:
            x = res

    return pooled3.reshape(B, D), attn, label3.reshape(B, 2)
